# baseline (speedup 1.0000x reference)
import functools

import jax
import jax.numpy as jnp
from jax import lax
from jax.experimental import pallas as pl
from jax.experimental.pallas import tpu as pltpu
from jax.experimental.pallas import tpu_sc as plsc

_D = 64
_BATCH = 16384
_MAXIDX = 1023
_NW = 32
_BPW = _BATCH // _NW
_CHUNK = 128
_NCHUNK = _BPW // _CHUNK
_LANES = 16
_C8 = _D // 8
_BLK = _NCHUNK * 8 * 128


@functools.partial(
    pl.kernel,
    out_type=jax.ShapeDtypeStruct((_BATCH * _D,), jnp.float32),
    mesh=plsc.VectorSubcoreMesh(core_axis_name="c", subcore_axis_name="s"),
    scratch_types=[
        pltpu.VMEM((_BPW,), jnp.float32),
        pltpu.VMEM((_NCHUNK, _CHUNK), jnp.int32),
        pltpu.VMEM((_BPW, _D), jnp.float32),
        pltpu.VMEM((_C8 * _BLK,), jnp.float32),
        pltpu.SemaphoreType.DMA,
        pltpu.SemaphoreType.DMA,
    ],
    compiler_params=pltpu.CompilerParams(
        use_tc_tiling_on_sc=False, needs_layout_passes=False
    ),
)
def _recency_gather(rec_hbm, table_hbm, out_hbm, rec_v, idx_v, rows_v, blk_v,
                    sem_g, sem_o):
    wid = lax.axis_index("s") * 2 + lax.axis_index("c")
    base = wid * _BPW
    pltpu.sync_copy(rec_hbm.at[pl.ds(base, _BPW)], rec_v)

    vecs_per_chunk = _CHUNK // _LANES
    gathers = []
    for j in range(_NCHUNK):
        for i in range(vecs_per_chunk):
            r = rec_v[pl.ds(j * _CHUNK + i * _LANES, _LANES)]
            ix = jnp.minimum((r * 0.5).astype(jnp.int32), _MAXIDX)
            idx_v[j, pl.ds(i * _LANES, _LANES)] = ix
        gathers.append(
            pltpu.async_copy(
                table_hbm.at[idx_v.at[j]],
                rows_v.at[pl.ds(j * _CHUNK, _CHUNK)],
                sem_g,
            )
        )

    iota = lax.iota(jnp.int32, _LANES)
    half = iota // 8
    low = iota - half * 8
    colpat = [half * _BLK + low * 128 + (2 * c0) * _BLK for c0 in range(_D // _LANES)]

    for k in range(_NCHUNK):
        gathers[k].wait()

        @plsc.parallel_loop(0, _CHUNK, unroll=16)
        def _tbody(d3, k=k):
            rbase = k * 1024 + d3
            for c0 in range(_D // _LANES):
                v = rows_v[_CHUNK * k + d3, pl.ds(_LANES * c0, _LANES)]
                plsc.store_scatter(blk_v, [colpat[c0] + rbase], v)

    writes = []
    for c8 in range(_C8):
        writes.append(
            pltpu.async_copy(
                blk_v.at[pl.ds(c8 * _BLK, _BLK)],
                out_hbm.at[pl.ds(c8 * (_BATCH * 8) + wid * _BLK, _BLK)],
                sem_o,
            )
        )
    for w in writes:
        w.wait()


def kernel(recency, table):
    x = _recency_gather(recency, table)
    x = x.reshape(_C8, _BATCH // 128, 8, 128)
    return x.transpose(1, 3, 0, 2).reshape(_BATCH, _D)


# speedup vs baseline: 1.0100x; 1.0100x over previous
import functools

import jax
import jax.numpy as jnp
from jax import lax
from jax.experimental import pallas as pl
from jax.experimental.pallas import tpu as pltpu
from jax.experimental.pallas import tpu_sc as plsc

_D = 64
_BATCH = 16384
_MAXIDX = 1023
_NW = 32
_BPW = _BATCH // _NW
_CHUNK = 128
_NCHUNK = _BPW // _CHUNK
_LANES = 16
_C8 = _D // 8
_BLK = _NCHUNK * 8 * 128


@functools.partial(
    pl.kernel,
    out_type=jax.ShapeDtypeStruct((_BATCH * _D,), jnp.float32),
    mesh=plsc.VectorSubcoreMesh(core_axis_name="c", subcore_axis_name="s"),
    scratch_types=[
        pltpu.VMEM((_BPW,), jnp.float32),
        pltpu.VMEM((_NCHUNK, _CHUNK), jnp.int32),
        pltpu.VMEM((_BPW, _D), jnp.float32),
        pltpu.VMEM((_C8 * _BLK,), jnp.float32),
        pltpu.SemaphoreType.DMA,
        pltpu.SemaphoreType.DMA,
    ],
    compiler_params=pltpu.CompilerParams(
        use_tc_tiling_on_sc=False, needs_layout_passes=False
    ),
)
def _recency_gather(rec_hbm, table_hbm, out_hbm, rec_v, idx_v, rows_v, blk_v,
                    sem_g, sem_o):
    wid = lax.axis_index("s") * 2 + lax.axis_index("c")
    base = wid * _BPW
    pltpu.sync_copy(rec_hbm.at[pl.ds(base, _BPW)], rec_v)

    vecs_per_chunk = _CHUNK // _LANES
    gathers = []
    for j in range(_NCHUNK):
        for i in range(vecs_per_chunk):
            r = rec_v[pl.ds(j * _CHUNK + i * _LANES, _LANES)]
            ix = jnp.minimum((r * 0.5).astype(jnp.int32), _MAXIDX)
            idx_v[j, pl.ds(i * _LANES, _LANES)] = ix
        gathers.append(
            pltpu.async_copy(
                table_hbm.at[idx_v.at[j]],
                rows_v.at[pl.ds(j * _CHUNK, _CHUNK)],
                sem_g,
            )
        )

    iota = lax.iota(jnp.int32, _LANES)
    half = iota // 8
    low = iota - half * 8
    colpat = [half * _BLK + low * 128 + (2 * c0) * _BLK for c0 in range(_D // _LANES)]

    one = jnp.full((_LANES,), 1, jnp.int32)
    for k in range(_NCHUNK):
        gathers[k].wait()

        @plsc.parallel_loop(
            0, _CHUNK, unroll=8,
            carry=tuple(p + k * 1024 for p in colpat),
        )
        def _tbody(d3, carry, k=k):
            for c0 in range(_D // _LANES):
                v = rows_v[_CHUNK * k + d3, pl.ds(_LANES * c0, _LANES)]
                plsc.store_scatter(blk_v, [carry[c0]], v)
            return tuple(c + one for c in carry)

    writes = []
    for c8 in range(_C8):
        writes.append(
            pltpu.async_copy(
                blk_v.at[pl.ds(c8 * _BLK, _BLK)],
                out_hbm.at[pl.ds(c8 * (_BATCH * 8) + wid * _BLK, _BLK)],
                sem_o,
            )
        )
    for w in writes:
        w.wait()


def kernel(recency, table):
    x = _recency_gather(recency, table)
    x = x.reshape(_C8, _BATCH // 128, 8, 128)
    return x.transpose(1, 3, 0, 2).reshape(_BATCH, _D)


# speedup vs baseline: 1.0163x; 1.0062x over previous
import functools

import jax
import jax.numpy as jnp
from jax import lax
from jax.experimental import pallas as pl
from jax.experimental.pallas import tpu as pltpu
from jax.experimental.pallas import tpu_sc as plsc

_D = 64
_BATCH = 16384
_MAXIDX = 1023
_NW = 32
_BPW = _BATCH // _NW
_CHUNK = 128
_NCHUNK = _BPW // _CHUNK
_LANES = 16
_C8 = _D // 8
_BLK = _NCHUNK * 8 * 128


@functools.partial(
    pl.kernel,
    out_type=jax.ShapeDtypeStruct((_BATCH * _D,), jnp.float32),
    mesh=plsc.VectorSubcoreMesh(core_axis_name="c", subcore_axis_name="s"),
    scratch_types=[
        pltpu.VMEM((_BPW,), jnp.float32),
        pltpu.VMEM((_NCHUNK, _CHUNK), jnp.int32),
        pltpu.VMEM((_BPW, _D), jnp.float32),
        pltpu.VMEM((_C8 * _BLK,), jnp.float32),
        pltpu.SemaphoreType.DMA,
        pltpu.SemaphoreType.DMA,
    ],
    compiler_params=pltpu.CompilerParams(
        use_tc_tiling_on_sc=False, needs_layout_passes=False
    ),
)
def _recency_gather(rec_hbm, table_hbm, out_hbm, rec_v, idx_v, rows_v, blk_v,
                    sem_g, sem_o):
    wid = lax.axis_index("s") * 2 + lax.axis_index("c")
    base = wid * _BPW
    pltpu.sync_copy(rec_hbm.at[pl.ds(base, _BPW)], rec_v)

    vecs_per_chunk = _CHUNK // _LANES
    gathers = []
    for j in range(_NCHUNK):
        for i in range(vecs_per_chunk):
            r = rec_v[pl.ds(j * _CHUNK + i * _LANES, _LANES)]
            ix = jnp.minimum((r * 0.5).astype(jnp.int32), _MAXIDX)
            idx_v[j, pl.ds(i * _LANES, _LANES)] = ix
        gathers.append(
            pltpu.async_copy(
                table_hbm.at[idx_v.at[j]],
                rows_v.at[pl.ds(j * _CHUNK, _CHUNK)],
                sem_g,
            )
        )

    iota = lax.iota(jnp.int32, _LANES)
    half = iota // 8
    low = iota - half * 8
    colpat = [half * _BLK + low * 128 + (2 * c0) * _BLK for c0 in range(_D // _LANES)]

    for k in range(_NCHUNK):
        gathers[k].wait()

        @plsc.parallel_loop(0, _CHUNK, unroll=8)
        def _tbody(d3, k=k):
            rbase = k * 1024 + d3
            for c0 in range(_D // _LANES):
                v = rows_v[_CHUNK * k + d3, pl.ds(_LANES * c0, _LANES)]
                plsc.store_scatter(blk_v, [colpat[c0] + rbase], v)

    writes = []
    for c8 in range(_C8):
        writes.append(
            pltpu.async_copy(
                blk_v.at[pl.ds(c8 * _BLK, _BLK)],
                out_hbm.at[pl.ds(c8 * (_BATCH * 8) + wid * _BLK, _BLK)],
                sem_o,
            )
        )
    for w in writes:
        w.wait()


def kernel(recency, table):
    x = _recency_gather(recency, table)
    x = x.reshape(_C8, _BATCH // 128, 8, 128)
    return x.transpose(1, 3, 0, 2).reshape(_BATCH, _D)
